# Initial kernel scaffold; baseline (speedup 1.0000x reference)
import jax
import jax.numpy as jnp
from jax.experimental import pallas as pl


def kernel(src, trg):
    raise NotImplementedError("write your pallas kernel here")


# speedup vs baseline: 2.5351x; 2.5351x over previous
import functools
import math

import jax
import jax.numpy as jnp
from jax import lax
from jax.experimental import pallas as pl
from jax.experimental.pallas import tpu as pltpu
from jax.experimental.pallas import tpu_sc as plsc

SIZE = 32000
N_ROWS = 4096
IGNORE_IDX = -100
SMOOTHING = 0.1
CONFIDENCE = 1.0 - SMOOTHING
IGN_COL = SIZE + IGNORE_IDX
EPS = SMOOTHING / (SIZE - 2)
_LOG_EPS = math.log(EPS)
ENT_A = (SIZE - 2) * EPS * _LOG_EPS + CONFIDENCE * math.log(CONFIDENCE)
ENT_B = (SIZE - 1) * EPS * _LOG_EPS

LANES = 16
ROW_WORDS = SIZE // LANES
NUM_WORKERS = 32
RPW = N_ROWS // NUM_WORKERS


def _sc_gather_body(src1, trg_hbm, g_out, trg_v, idx_v, val_v, sem):
    wid = lax.axis_index("s") * 2 + lax.axis_index("c")
    base = wid * RPW
    pltpu.sync_copy(trg_hbm.at[pl.ds(base, RPW)], trg_v)
    iota = lax.iota(jnp.int32, LANES)
    for k in range(RPW // LANES):
        t = jnp.maximum(trg_v[pl.ds(k * LANES, LANES)], 0)
        rows = (base + k * LANES) + iota
        idx_v[pl.ds(k * LANES, LANES)] = rows * SIZE + t
    pltpu.async_copy(src1.at[idx_v], val_v, sem).wait()
    pltpu.sync_copy(val_v, g_out.at[pl.ds(base, RPW)])


def _sc_gather(src1, trg32):
    mesh = plsc.VectorSubcoreMesh(core_axis_name="c", subcore_axis_name="s")
    f = functools.partial(
        pl.kernel,
        mesh=mesh,
        out_type=jax.ShapeDtypeStruct((N_ROWS,), jnp.float32),
        scratch_types=[
            pltpu.VMEM((RPW,), jnp.int32),
            pltpu.VMEM((RPW,), jnp.int32),
            pltpu.VMEM((RPW,), jnp.float32),
            pltpu.SemaphoreType.DMA,
        ],
    )(_sc_gather_body)
    return f(src1, trg32)


BR = 512
BC = 6400
GR = N_ROWS // BR
GC = SIZE // BC
C_BLOCK = IGN_COL // BC
C_LOCAL = IGN_COL - C_BLOCK * BC


def _rowsum_body(src_ref, rs_ref, cv_ref):
    c = pl.program_id(1)
    part = jnp.sum(src_ref[...], axis=1, keepdims=True)

    @pl.when(c == 0)
    def _():
        rs_ref[...] = part

    @pl.when(c != 0)
    def _():
        rs_ref[...] = rs_ref[...] + part

    @pl.when(c == C_BLOCK)
    def _():
        cv_ref[...] = src_ref[:, C_LOCAL:C_LOCAL + 1]


def _rowsum(src):
    return pl.pallas_call(
        _rowsum_body,
        grid=(GR, GC),
        in_specs=[pl.BlockSpec((BR, BC), lambda r, c: (r, c))],
        out_specs=[
            pl.BlockSpec((BR, 1), lambda r, c: (r, 0)),
            pl.BlockSpec((BR, 1), lambda r, c: (r, 0)),
        ],
        out_shape=[
            jax.ShapeDtypeStruct((N_ROWS, 1), jnp.float32),
            jax.ShapeDtypeStruct((N_ROWS, 1), jnp.float32),
        ],
    )(src)


CR = 32
CC = N_ROWS // CR


def _combine_body(rs_ref, cv_ref, g_ref, trg_ref, out_ref):
    rs = rs_ref[...]
    cv = cv_ref[...]
    g = g_ref[...]
    t = trg_ref[...]
    contrib_a = ENT_A - EPS * (rs - g - cv) - CONFIDENCE * g
    contrib_b = ENT_B - EPS * (rs - cv)
    contrib = jnp.where(t == IGN_COL, contrib_b, contrib_a)
    contrib = jnp.where(t == IGNORE_IDX, 0.0, contrib)
    out_ref[...] = (jnp.sum(contrib) / N_ROWS).reshape(1, 1)


def _combine(rs, cv, g, trg32):
    return pl.pallas_call(
        _combine_body,
        out_shape=jax.ShapeDtypeStruct((1, 1), jnp.float32),
    )(rs.reshape(CR, CC), cv.reshape(CR, CC), g.reshape(CR, CC),
      trg32.reshape(CR, CC))


def kernel(src, trg):
    trg32 = trg.astype(jnp.int32)
    g = _sc_gather(src.reshape(-1), trg32)
    rs, cv = _rowsum(src)
    return _combine(rs, cv, g, trg32)[0, 0]
